# baseline (speedup 1.0000x reference)
import functools

import jax
import jax.numpy as jnp
from jax import lax
from jax.experimental import pallas as pl
from jax.experimental.pallas import tpu as pltpu
from jax.experimental.pallas import tpu_sc as plsc

_N, _E, _D = 10000, 320000, 128
_NC, _NS = 2, 16
_C = 128
_CPT = 80
_CPH = 40
_EPAD = _NC * _NS * _CPT * _C
_NPAD = 10240
_RPT = _NPAD // _NS
_BLK = 1000
_G = _N // _BLK


def _dot_t(x, w):
    return lax.dot_general(x.astype(jnp.bfloat16), w.astype(jnp.bfloat16),
                           (((1,), (1,)), ((), ())),
                           preferred_element_type=jnp.float32)


def _tc_a_body(x_ref, px_ref, wq_ref, bq_ref, wk_ref, bk_ref, h_ref):
    q = _dot_t(x_ref[...], wq_ref[...]) + bq_ref[...]
    k = _dot_t(px_ref[...], wk_ref[...]) + bk_ref[...]
    h_ref[...] = jnp.maximum(q * k, 0.0)


def _tc_b_body(h_ref, px_ref, a_ref, x_ref, wv_ref, bv_ref, wh_ref, bh_ref,
               w1_ref, b1_ref, w2_ref, b2_ref, o_ref):
    v = _dot_t(px_ref[...], wv_ref[...]) + bv_ref[...]
    t = h_ref[...] + v + a_ref[...]
    ho = jnp.maximum(_dot_t(t, wh_ref[...]) + bh_ref[...], 0.0) + x_ref[...]
    h1 = jnp.maximum(_dot_t(ho, w1_ref[...]) + b1_ref[...], 0.0)
    o_ref[...] = _dot_t(h1, w2_ref[...]) + b2_ref[...]


def _row_spec(d):
    return pl.BlockSpec((_BLK, d), lambda i: (i, 0))


def _full_spec(r, c):
    return pl.BlockSpec((r, c), lambda i: (0, 0))


def _tc_a(x, px, wq, bq, wk, bk):
    return pl.pallas_call(
        _tc_a_body,
        grid=(_G,),
        in_specs=[_row_spec(_D), _row_spec(_D),
                  _full_spec(_D, _D), _full_spec(1, _D),
                  _full_spec(_D, _D), _full_spec(1, _D)],
        out_specs=_row_spec(_D),
        out_shape=jax.ShapeDtypeStruct((_N, _D), jnp.float32),
    )(x, px, wq, bq, wk, bk)


def _tc_b(h, px, aggr, x, wv, bv, wh, bh, w1, b1, w2, b2):
    return pl.pallas_call(
        _tc_b_body,
        grid=(_G,),
        in_specs=[_row_spec(_D), _row_spec(_D),
                  _row_spec(_D),
                  _row_spec(_D),
                  _full_spec(_D, _D), _full_spec(1, _D),
                  _full_spec(_D, _D), _full_spec(1, _D),
                  _full_spec(2 * _D, _D), _full_spec(1, 2 * _D),
                  _full_spec(_D, 2 * _D), _full_spec(1, _D)],
        out_specs=_row_spec(_D),
        out_shape=jax.ShapeDtypeStruct((_N, _D), jnp.float32),
    )(h, px, aggr, x, wv, bv, wh, bh, w1, b1, w2, b2)


def _sc_body(h_hbm, src_hbm, dst_hbm, zeros_hbm, out_hbm,
             idx_s, idx_d, rows0, rows1, acc, sem0, sem1):
    c = lax.axis_index("c")
    s = lax.axis_index("s")
    wid = s * _NC + c
    r0 = pl.multiple_of(s * _RPT, 8)
    pltpu.sync_copy(zeros_hbm, acc.at[pl.ds(r0, _RPT)])
    base = pl.multiple_of(wid * _CPT, 8)
    pltpu.sync_copy(src_hbm.at[pl.ds(base, _CPT)], idx_s)
    pltpu.sync_copy(dst_hbm.at[pl.ds(base, _CPT)], idx_d)
    plsc.subcore_barrier()

    pltpu.async_copy(h_hbm.at[idx_s.at[0]], rows0, sem0)

    @pl.loop(0, _CPT, step=2)
    def _pair(i):
        pltpu.async_copy(h_hbm.at[idx_s.at[i + 1]], rows1, sem1)
        pltpu.make_async_copy(h_hbm.at[idx_s.at[i]], rows0, sem0).wait()
        pltpu.sync_copy(rows0, acc.at[idx_d.at[i]], add=True)

        @pl.when(i + 2 < _CPT)
        def _():
            pltpu.async_copy(h_hbm.at[idx_s.at[i + 2]], rows0, sem0)

        pltpu.make_async_copy(h_hbm.at[idx_s.at[i + 1]], rows1, sem1).wait()
        pltpu.sync_copy(rows1, acc.at[idx_d.at[i + 1]], add=True)

    plsc.subcore_barrier()
    pltpu.sync_copy(acc.at[pl.ds(r0, _RPT)], out_hbm.at[c, pl.ds(r0, _RPT)])


@functools.cache
def _sc_agg_fn():
    return pl.kernel(
        _sc_body,
        out_type=jax.ShapeDtypeStruct((_NC, _NPAD, _D), jnp.bfloat16),
        mesh=plsc.VectorSubcoreMesh(core_axis_name="c", subcore_axis_name="s",
                                    num_cores=_NC, num_subcores=_NS),
        compiler_params=pltpu.CompilerParams(use_tc_tiling_on_sc=False),
        scratch_types=[
            pltpu.VMEM((_CPT, _C), jnp.int32),
            pltpu.VMEM((_CPT, _C), jnp.int32),
            pltpu.VMEM((_C, _D), jnp.bfloat16),
            pltpu.VMEM((_C, _D), jnp.bfloat16),
            pltpu.VMEM_SHARED((_NPAD, _D), jnp.bfloat16),
            pltpu.SemaphoreType.DMA,
            pltpu.SemaphoreType.DMA,
        ],
    )


def kernel(x, prompt_x, edge_index, Wq, bq, Wk, bk, Wv, bv, Wh, bh, W1, b1, W2, b2):
    npad = _EPAD - _E
    pad_iota = jnp.arange(npad, dtype=jnp.int32)
    src = jnp.concatenate([edge_index[0].astype(jnp.int32), pad_iota % _N]
                          ).reshape(_NC * _NS * _CPT, _C)
    dst = jnp.concatenate([edge_index[1].astype(jnp.int32),
                           _N + pad_iota % (_NPAD - _N)]
                          ).reshape(_NC * _NS * _CPT, _C)
    h = _tc_a(x, prompt_x, Wq, bq.reshape(1, _D), Wk, bk.reshape(1, _D))
    zeros = jnp.zeros((_RPT, _D), jnp.bfloat16)
    aggr = _sc_agg_fn()(h.astype(jnp.bfloat16), src, dst, zeros)
    aggr_sum = (aggr[0, :_N].astype(jnp.float32)
                + aggr[1, :_N].astype(jnp.float32))
    return _tc_b(h, prompt_x, aggr_sum, x, Wv, bv.reshape(1, _D),
                 Wh, bh.reshape(1, _D), W1, b1.reshape(1, 2 * _D),
                 W2, b2.reshape(1, _D))


# speedup vs baseline: 1.0094x; 1.0094x over previous
import functools

import jax
import jax.numpy as jnp
from jax import lax
from jax.experimental import pallas as pl
from jax.experimental.pallas import tpu as pltpu
from jax.experimental.pallas import tpu_sc as plsc

_N, _E, _D = 10000, 320000, 128
_NC, _NS = 2, 16
_C = 128
_CPT = 80
_CPH = 40
_EPAD = _NC * _NS * _CPT * _C
_NPAD = 10240
_RPT = _NPAD // _NS
_BLK = 1000
_G = _N // _BLK


def _dot_t(x, w):
    return lax.dot_general(x.astype(jnp.bfloat16), w.astype(jnp.bfloat16),
                           (((1,), (1,)), ((), ())),
                           preferred_element_type=jnp.float32)


def _tc_a_body(x_ref, px_ref, wq_ref, bq_ref, wk_ref, bk_ref, h_ref, h16_ref):
    q = _dot_t(x_ref[...], wq_ref[...]) + bq_ref[...]
    k = _dot_t(px_ref[...], wk_ref[...]) + bk_ref[...]
    h = jnp.maximum(q * k, 0.0)
    h_ref[...] = h
    h16_ref[...] = h.astype(jnp.bfloat16)


def _tc_b_body(h_ref, px_ref, a_ref, x_ref, wv_ref, bv_ref, wh_ref, bh_ref,
               w1_ref, b1_ref, w2_ref, b2_ref, o_ref):
    v = _dot_t(px_ref[...], wv_ref[...]) + bv_ref[...]
    t = h_ref[...] + v + a_ref[...]
    ho = jnp.maximum(_dot_t(t, wh_ref[...]) + bh_ref[...], 0.0) + x_ref[...]
    h1 = jnp.maximum(_dot_t(ho, w1_ref[...]) + b1_ref[...], 0.0)
    o_ref[...] = _dot_t(h1, w2_ref[...]) + b2_ref[...]


def _row_spec(d):
    return pl.BlockSpec((_BLK, d), lambda i: (i, 0))


def _full_spec(r, c):
    return pl.BlockSpec((r, c), lambda i: (0, 0))


def _tc_a(x, px, wq, bq, wk, bk):
    return pl.pallas_call(
        _tc_a_body,
        grid=(_G,),
        in_specs=[_row_spec(_D), _row_spec(_D),
                  _full_spec(_D, _D), _full_spec(1, _D),
                  _full_spec(_D, _D), _full_spec(1, _D)],
        out_specs=[_row_spec(_D), _row_spec(_D)],
        out_shape=[jax.ShapeDtypeStruct((_N, _D), jnp.float32),
                   jax.ShapeDtypeStruct((_N, _D), jnp.bfloat16)],
    )(x, px, wq, bq, wk, bk)


def _tc_b(h, px, aggr, x, wv, bv, wh, bh, w1, b1, w2, b2):
    return pl.pallas_call(
        _tc_b_body,
        grid=(_G,),
        in_specs=[_row_spec(_D), _row_spec(_D),
                  _row_spec(_D),
                  _row_spec(_D),
                  _full_spec(_D, _D), _full_spec(1, _D),
                  _full_spec(_D, _D), _full_spec(1, _D),
                  _full_spec(2 * _D, _D), _full_spec(1, 2 * _D),
                  _full_spec(_D, 2 * _D), _full_spec(1, _D)],
        out_specs=_row_spec(_D),
        out_shape=jax.ShapeDtypeStruct((_N, _D), jnp.float32),
    )(h, px, aggr, x, wv, bv, wh, bh, w1, b1, w2, b2)


def _sc_body(h_hbm, src_hbm, dst_hbm, zeros_hbm, out_hbm,
             idx_s, idx_d, rows0, rows1, acc, sem0, sem1):
    c = lax.axis_index("c")
    s = lax.axis_index("s")
    wid = s * _NC + c
    r0 = pl.multiple_of(s * _RPT, 8)
    pltpu.sync_copy(zeros_hbm, acc.at[pl.ds(r0, _RPT)])
    base = pl.multiple_of(wid * _CPT, 8)
    pltpu.sync_copy(src_hbm.at[pl.ds(base, _CPT)], idx_s)
    pltpu.sync_copy(dst_hbm.at[pl.ds(base, _CPT)], idx_d)
    plsc.subcore_barrier()

    pltpu.async_copy(h_hbm.at[idx_s.at[0]], rows0, sem0)

    @pl.loop(0, _CPT, step=2)
    def _pair(i):
        pltpu.async_copy(h_hbm.at[idx_s.at[i + 1]], rows1, sem1)
        pltpu.make_async_copy(h_hbm.at[idx_s.at[i]], rows0, sem0).wait()
        pltpu.sync_copy(rows0, acc.at[idx_d.at[i]], add=True)

        @pl.when(i + 2 < _CPT)
        def _():
            pltpu.async_copy(h_hbm.at[idx_s.at[i + 2]], rows0, sem0)

        pltpu.make_async_copy(h_hbm.at[idx_s.at[i + 1]], rows1, sem1).wait()
        pltpu.sync_copy(rows1, acc.at[idx_d.at[i + 1]], add=True)

    plsc.subcore_barrier()
    pltpu.sync_copy(acc.at[pl.ds(r0, _RPT)], out_hbm.at[c, pl.ds(r0, _RPT)])


@functools.cache
def _sc_agg_fn():
    return pl.kernel(
        _sc_body,
        out_type=jax.ShapeDtypeStruct((_NC, _NPAD, _D), jnp.bfloat16),
        mesh=plsc.VectorSubcoreMesh(core_axis_name="c", subcore_axis_name="s",
                                    num_cores=_NC, num_subcores=_NS),
        compiler_params=pltpu.CompilerParams(use_tc_tiling_on_sc=False),
        scratch_types=[
            pltpu.VMEM((_CPT, _C), jnp.int32),
            pltpu.VMEM((_CPT, _C), jnp.int32),
            pltpu.VMEM((_C, _D), jnp.bfloat16),
            pltpu.VMEM((_C, _D), jnp.bfloat16),
            pltpu.VMEM_SHARED((_NPAD, _D), jnp.bfloat16),
            pltpu.SemaphoreType.DMA,
            pltpu.SemaphoreType.DMA,
        ],
    )


def kernel(x, prompt_x, edge_index, Wq, bq, Wk, bk, Wv, bv, Wh, bh, W1, b1, W2, b2):
    npad = _EPAD - _E
    pad_iota = jnp.arange(npad, dtype=jnp.int32)
    src = jnp.concatenate([edge_index[0].astype(jnp.int32), pad_iota % _N]
                          ).reshape(_NC * _NS * _CPT, _C)
    dst = jnp.concatenate([edge_index[1].astype(jnp.int32),
                           _N + pad_iota % (_NPAD - _N)]
                          ).reshape(_NC * _NS * _CPT, _C)
    h, h16 = _tc_a(x, prompt_x, Wq, bq.reshape(1, _D), Wk, bk.reshape(1, _D))
    zeros = jnp.zeros((_RPT, _D), jnp.bfloat16)
    aggr = _sc_agg_fn()(h16, src, dst, zeros)
    aggr_sum = (aggr[0, :_N].astype(jnp.float32)
                + aggr[1, :_N].astype(jnp.float32))
    return _tc_b(h, prompt_x, aggr_sum, x, Wv, bv.reshape(1, _D),
                 Wh, bh.reshape(1, _D), W1, b1.reshape(1, 2 * _D),
                 W2, b2.reshape(1, _D))
